# baseline (speedup 1.0000x reference)
import functools

import jax
import jax.numpy as jnp
from jax import lax
from jax.experimental import pallas as pl
from jax.experimental.pallas import tpu as pltpu
from jax.experimental.pallas import tpu_sc as plsc

_N_NODES = 50000
_D = 256
_E = 256
_B = 10000
_K = 10

_NS = 16

_DW = _D // 2
_BP = 10240
_BPW = _BP // _NS
_CH = 16
_NCH = _BPW // _CH
_SCH = 40
_NSCH = _BPW // _SCH

_mesh = plsc.VectorSubcoreMesh(
    core_axis_name="c", subcore_axis_name="s", num_cores=1, num_subcores=_NS
)


@functools.partial(
    pl.kernel,
    out_type=[
        jax.ShapeDtypeStruct((_BP, _DW), jnp.int32),
        jax.ShapeDtypeStruct((_BP, _DW), jnp.int32),
    ],
    mesh=_mesh,
    scratch_types=[
        pltpu.VMEM((_CH * _K,), jnp.int32),
        pltpu.VMEM((_CH * _K,), jnp.int32),
        pltpu.VMEM((_SCH,), jnp.int32),
        pltpu.VMEM((_SCH,), jnp.int32),
        pltpu.VMEM((_CH * _K, _DW), jnp.int32),
        pltpu.VMEM((_CH * _K, _DW), jnp.int32),
        pltpu.VMEM((_CH, _DW), jnp.int32),
        pltpu.VMEM((_CH, _DW), jnp.int32),
        pltpu.VMEM((_SCH, _DW), jnp.int32),
        pltpu.VMEM((_SCH, _DW), jnp.int32),
        pltpu.SemaphoreType.DMA,
        pltpu.SemaphoreType.DMA,
        pltpu.SemaphoreType.DMA,
        pltpu.SemaphoreType.DMA,
        pltpu.SemaphoreType.DMA,
        pltpu.SemaphoreType.DMA,
        pltpu.SemaphoreType.DMA,
        pltpu.SemaphoreType.DMA,
    ],
)
def _sc_gather(features, nodes, neigh, self_out, neigh_out,
               cidxA, cidxB, scidxA, scidxB,
               nbufA, nbufB, obufA, obufB, sbufA, sbufB,
               semA, semB, semOA, semOB, semSA, semSB, semSOA, semSOB):
    sid = lax.axis_index("s")
    base = sid * _BPW

    def ngather(c, cidx, buf, sem):
        pltpu.sync_copy(neigh.at[pl.ds(base * _K + c * _CH * _K, _CH * _K)], cidx)
        pltpu.make_async_copy(features.at[cidx], buf, sem).start()

    def nwait(cidx, buf, sem):
        pltpu.make_async_copy(features.at[cidx], buf, sem).wait()

    def _bc(x, dt):
        return jax.lax.bitcast_convert_type(x, dt)

    hi_mask = jnp.full((16,), 0xFFFF0000, dtype=jnp.uint32)
    half_ulp = jnp.full((16,), 0x8000, dtype=jnp.uint32)
    sixteen = jnp.full((16,), 16, dtype=jnp.uint32)

    def reduce_chunk(buf, obuf):
        def row_body(r, rcarry):
            rk = r * _K
            for j in range(_DW // 16):
                sl = pl.ds(j * 16, 16)
                v = _bc(buf[rk, sl], jnp.uint32)
                os_ = _bc(v & hi_mask, jnp.float32)
                es = _bc(v << sixteen, jnp.float32)
                for t in range(1, _K):
                    v = _bc(buf[rk + t, sl], jnp.uint32)
                    os_ = os_ + _bc(v & hi_mask, jnp.float32)
                    es = es + _bc(v << sixteen, jnp.float32)
                eb = (_bc(es, jnp.uint32) + half_ulp) >> sixteen
                ob = (_bc(os_, jnp.uint32) + half_ulp) & hi_mask
                obuf[r, sl] = _bc(eb | ob, jnp.int32)
            return rcarry

        lax.fori_loop(0, _CH, row_body, 0, unroll=False)

    def out_start(c, obuf, sem):
        pltpu.make_async_copy(
            obuf, neigh_out.at[pl.ds(base + c * _CH, _CH), :], sem
        ).start()

    def out_wait(obuf, sem):
        pltpu.make_async_copy(
            obuf, neigh_out.at[pl.ds(base, _CH), :], sem
        ).wait()

    def sgather(c, scidx, buf, sem):
        pltpu.sync_copy(nodes.at[pl.ds(base + c * _SCH, _SCH)], scidx)
        pltpu.make_async_copy(features.at[scidx], buf, sem).start()

    ngather(0, cidxA, nbufA, semA)
    ngather(1, cidxB, nbufB, semB)

    scidxs = (scidxA, scidxB)
    sbufs = (sbufA, sbufB)
    ssems = (semSA, semSB)
    sosems = (semSOA, semSOB)
    sgather(0, scidxA, sbufA, semSA)
    sgather(1, scidxB, sbufB, semSB)
    for c in range(_NSCH):
        x = c % 2
        pltpu.make_async_copy(features.at[scidxs[x]], sbufs[x], ssems[x]).wait()
        dst = self_out.at[pl.ds(base + c * _SCH, _SCH), :]
        pltpu.make_async_copy(sbufs[x], dst, sosems[x]).start()
        if c + 2 < _NSCH:
            pltpu.make_async_copy(sbufs[x], dst, sosems[x]).wait()
            sgather(c + 2, scidxs[x], sbufs[x], ssems[x])
    for x in range(2):
        pltpu.make_async_copy(
            sbufs[x], self_out.at[pl.ds(base, _SCH), :], sosems[x]
        ).wait()

    def half(c, cnext, cidx, nbuf, obuf, sem, semO, g):
        nwait(cidx, nbuf, sem)
        pl.when(g > 0)(lambda: out_wait(obuf, semO))
        reduce_chunk(nbuf, obuf)
        out_start(c, obuf, semO)
        pl.when(cnext < _NCH)(lambda: ngather(cnext, cidx, nbuf, sem))

    def body(g, carry):
        c0 = 2 * g
        half(c0, c0 + 2, cidxA, nbufA, obufA, semA, semOA, g)
        half(c0 + 1, c0 + 3, cidxB, nbufB, obufB, semB, semOB, g)
        return carry

    lax.fori_loop(0, _NCH // 2, body, 0, unroll=False)
    out_wait(obufA, semOA)
    out_wait(obufB, semOB)


def _rn16(u):
    return u + jnp.uint32(0x7FFF) + ((u >> jnp.uint32(16)) & jnp.uint32(1))


def _pack_body(x_ref, o_ref):
    u = jax.lax.bitcast_convert_type(x_ref[...], jnp.uint32)
    r = _rn16(u)
    lo = r[:, :_DW] >> jnp.uint32(16)
    hi = r[:, _DW:] & jnp.uint32(0xFFFF0000)
    o_ref[...] = jax.lax.bitcast_convert_type(lo | hi, jnp.int32)


_BMP = 2000


def _pack_table(features):
    return pl.pallas_call(
        _pack_body,
        grid=(_N_NODES // _BMP,),
        in_specs=[pl.BlockSpec((_BMP, _D), lambda i: (i, 0))],
        out_specs=pl.BlockSpec((_BMP, _DW), lambda i: (i, 0)),
        out_shape=jax.ShapeDtypeStruct((_N_NODES, _DW), jnp.int32),
    )(features)


def _unpack_f32(w_i32):
    u = jax.lax.bitcast_convert_type(w_i32, jnp.uint32)
    lo = jax.lax.bitcast_convert_type(u << jnp.uint32(16), jnp.float32)
    hi = jax.lax.bitcast_convert_type(u & jnp.uint32(0xFFFF0000), jnp.float32)
    return lo, hi


def _mm_body(self_ref, neigh_ref, wsLo_ref, wsHi_ref, wnLo_ref, wnHi_ref, o_ref):
    slo, shi = _unpack_f32(self_ref[...])
    nlo, nhi = _unpack_f32(neigh_ref[...])
    acc = jnp.dot(slo, wsLo_ref[...], preferred_element_type=jnp.float32)
    acc += jnp.dot(shi, wsHi_ref[...], preferred_element_type=jnp.float32)
    acc += jnp.dot(nlo, wnLo_ref[...], preferred_element_type=jnp.float32)
    acc += jnp.dot(nhi, wnHi_ref[...], preferred_element_type=jnp.float32)
    o_ref[...] = jnp.maximum(acc, 0.0)


_BM = 1024


def _tc_combine(self_w, neigh_w, wsLo, wsHi, wnLo, wnHi):
    wspec = pl.BlockSpec((_DW, _E), lambda i: (0, 0))
    return pl.pallas_call(
        _mm_body,
        grid=(_BP // _BM,),
        in_specs=[
            pl.BlockSpec((_BM, _DW), lambda i: (i, 0)),
            pl.BlockSpec((_BM, _DW), lambda i: (i, 0)),
            wspec, wspec, wspec, wspec,
        ],
        out_specs=pl.BlockSpec((_BM, _E), lambda i: (i, 0)),
        out_shape=jax.ShapeDtypeStruct((_BP, _E), jnp.float32),
    )(self_w, neigh_w, wsLo, wsHi, wnLo, wnHi)


def kernel(features, nodes, neigh_idx, W):
    features_w = _pack_table(features)
    nodes_p = jnp.pad(nodes, (0, _BP - _B))
    neigh_p = jnp.pad(neigh_idx, ((0, _BP - _B), (0, 0))).reshape(_BP * _K)
    self_w, neigh_w = _sc_gather(features_w, nodes_p, neigh_p)
    wsT = W[:, :_D].T
    wnT = W[:, _D:].T * (1.0 / _K)
    out = _tc_combine(self_w, neigh_w,
                      wsT[:_DW], wsT[_DW:], wnT[:_DW], wnT[_DW:])
    return out[:_B]


# speedup vs baseline: 1.1883x; 1.1883x over previous
import functools

import jax
import jax.numpy as jnp
from jax import lax
from jax.experimental import pallas as pl
from jax.experimental.pallas import tpu as pltpu
from jax.experimental.pallas import tpu_sc as plsc

_N_NODES = 50000
_D = 256
_E = 256
_B = 10000
_K = 10

_NC = 2
_NS = 16

_DW = _D // 2
_BP = 10240
_BPP = _BP // _NS
_BPW0 = 448
_BPW1 = _BPP - _BPW0
_CH = 16
_SCH = 32
_NSCH_MAX = max(_BPW0, _BPW1) // _SCH

_mesh = plsc.VectorSubcoreMesh(
    core_axis_name="c", subcore_axis_name="s", num_cores=_NC, num_subcores=_NS
)


@functools.partial(
    pl.kernel,
    out_type=[
        jax.ShapeDtypeStruct((_BP, _DW), jnp.int32),
        jax.ShapeDtypeStruct((_BP, _DW), jnp.int32),
    ],
    mesh=_mesh,
    scratch_types=[
        pltpu.VMEM((_CH * _K,), jnp.int32),
        pltpu.VMEM((_CH * _K,), jnp.int32),
        pltpu.VMEM((_SCH,), jnp.int32),
        pltpu.VMEM((_SCH,), jnp.int32),
        pltpu.VMEM((_CH * _K, _DW), jnp.int32),
        pltpu.VMEM((_CH * _K, _DW), jnp.int32),
        pltpu.VMEM((_CH, _DW), jnp.int32),
        pltpu.VMEM((_CH, _DW), jnp.int32),
        pltpu.VMEM((_SCH, _DW), jnp.int32),
        pltpu.VMEM((_SCH, _DW), jnp.int32),
        pltpu.SemaphoreType.DMA,
        pltpu.SemaphoreType.DMA,
        pltpu.SemaphoreType.DMA,
        pltpu.SemaphoreType.DMA,
        pltpu.SemaphoreType.DMA,
        pltpu.SemaphoreType.DMA,
        pltpu.SemaphoreType.DMA,
        pltpu.SemaphoreType.DMA,
    ],
)
def _sc_gather(features, nodes, neigh, self_out, neigh_out,
               cidxA, cidxB, scidxA, scidxB,
               nbufA, nbufB, obufA, obufB, sbufA, sbufB,
               semA, semB, semOA, semOB, semSA, semSB, semSOA, semSOB):
    cid = lax.axis_index("c")
    sid = lax.axis_index("s")
    rows = jnp.where(cid == 0, _BPW0, _BPW1)
    base = cid * (_NS * _BPW0) + sid * rows
    nch = rows // _CH
    nsch = rows // _SCH

    def ngather(c, cidx, buf, sem):
        pltpu.sync_copy(neigh.at[pl.ds(base * _K + c * _CH * _K, _CH * _K)], cidx)
        pltpu.make_async_copy(features.at[cidx], buf, sem).start()

    def nwait(cidx, buf, sem):
        pltpu.make_async_copy(features.at[cidx], buf, sem).wait()

    def _bc(x, dt):
        return jax.lax.bitcast_convert_type(x, dt)

    hi_mask = jnp.full((16,), 0xFFFF0000, dtype=jnp.uint32)
    half_ulp = jnp.full((16,), 0x8000, dtype=jnp.uint32)
    sixteen = jnp.full((16,), 16, dtype=jnp.uint32)

    def reduce_chunk(buf, obuf):
        def row_body(r, rcarry):
            rk = r * _K
            for j in range(_DW // 16):
                sl = pl.ds(j * 16, 16)
                v = _bc(buf[rk, sl], jnp.uint32)
                os_ = _bc(v & hi_mask, jnp.float32)
                es = _bc(v << sixteen, jnp.float32)
                for t in range(1, _K):
                    v = _bc(buf[rk + t, sl], jnp.uint32)
                    os_ = os_ + _bc(v & hi_mask, jnp.float32)
                    es = es + _bc(v << sixteen, jnp.float32)
                eb = (_bc(es, jnp.uint32) + half_ulp) >> sixteen
                ob = (_bc(os_, jnp.uint32) + half_ulp) & hi_mask
                obuf[r, sl] = _bc(eb | ob, jnp.int32)
            return rcarry

        lax.fori_loop(0, _CH, row_body, 0, unroll=False)

    def out_start(c, obuf, sem):
        pltpu.make_async_copy(
            obuf, neigh_out.at[pl.ds(base + c * _CH, _CH), :], sem
        ).start()

    def out_wait(obuf, sem):
        pltpu.make_async_copy(
            obuf, neigh_out.at[pl.ds(base, _CH), :], sem
        ).wait()

    def sgather(c, scidx, buf, sem):
        pltpu.sync_copy(nodes.at[pl.ds(base + c * _SCH, _SCH)], scidx)
        pltpu.make_async_copy(features.at[scidx], buf, sem).start()

    ngather(0, cidxA, nbufA, semA)
    ngather(1, cidxB, nbufB, semB)

    scidxs = (scidxA, scidxB)
    sbufs = (sbufA, sbufB)
    ssems = (semSA, semSB)
    sosems = (semSOA, semSOB)
    sgather(0, scidxA, sbufA, semSA)
    pl.when(nsch > 1)(lambda: sgather(1, scidxB, sbufB, semSB))
    for c in range(_NSCH_MAX):
        x = c % 2

        def _sbody(c=c, x=x):
            pltpu.make_async_copy(features.at[scidxs[x]], sbufs[x], ssems[x]).wait()
            dst = self_out.at[pl.ds(base + c * _SCH, _SCH), :]
            pltpu.make_async_copy(sbufs[x], dst, sosems[x]).start()

        def _snext(c=c, x=x):
            dst = self_out.at[pl.ds(base + c * _SCH, _SCH), :]
            pltpu.make_async_copy(sbufs[x], dst, sosems[x]).wait()
            sgather(c + 2, scidxs[x], sbufs[x], ssems[x])

        pl.when(c < nsch)(_sbody)
        pl.when(c + 2 < nsch)(_snext)
    pltpu.make_async_copy(
        sbufs[0], self_out.at[pl.ds(base, _SCH), :], sosems[0]
    ).wait()
    pl.when(nsch > 1)(lambda: pltpu.make_async_copy(
        sbufs[1], self_out.at[pl.ds(base, _SCH), :], sosems[1]
    ).wait())

    def half(c, cnext, cidx, nbuf, obuf, sem, semO, g):
        nwait(cidx, nbuf, sem)
        pl.when(g > 0)(lambda: out_wait(obuf, semO))
        reduce_chunk(nbuf, obuf)
        out_start(c, obuf, semO)
        pl.when(cnext < nch)(lambda: ngather(cnext, cidx, nbuf, sem))

    def body(g, carry):
        c0 = 2 * g
        half(c0, c0 + 2, cidxA, nbufA, obufA, semA, semOA, g)
        half(c0 + 1, c0 + 3, cidxB, nbufB, obufB, semB, semOB, g)
        return carry

    lax.fori_loop(0, nch // 2, body, 0, unroll=False)
    out_wait(obufA, semOA)
    out_wait(obufB, semOB)


def _rn16(u):
    return u + jnp.uint32(0x7FFF) + ((u >> jnp.uint32(16)) & jnp.uint32(1))


def _pack_body(x_ref, o_ref):
    u = jax.lax.bitcast_convert_type(x_ref[...], jnp.uint32)
    r = _rn16(u)
    lo = r[:, :_DW] >> jnp.uint32(16)
    hi = r[:, _DW:] & jnp.uint32(0xFFFF0000)
    o_ref[...] = jax.lax.bitcast_convert_type(lo | hi, jnp.int32)


_BMP = 2000


def _pack_table(features):
    return pl.pallas_call(
        _pack_body,
        grid=(_N_NODES // _BMP,),
        in_specs=[pl.BlockSpec((_BMP, _D), lambda i: (i, 0))],
        out_specs=pl.BlockSpec((_BMP, _DW), lambda i: (i, 0)),
        out_shape=jax.ShapeDtypeStruct((_N_NODES, _DW), jnp.int32),
    )(features)


def _unpack_f32(w_i32):
    u = jax.lax.bitcast_convert_type(w_i32, jnp.uint32)
    lo = jax.lax.bitcast_convert_type(u << jnp.uint32(16), jnp.float32)
    hi = jax.lax.bitcast_convert_type(u & jnp.uint32(0xFFFF0000), jnp.float32)
    return lo, hi


def _mm_body(self_ref, neigh_ref, wsLo_ref, wsHi_ref, wnLo_ref, wnHi_ref, o_ref):
    slo, shi = _unpack_f32(self_ref[...])
    nlo, nhi = _unpack_f32(neigh_ref[...])
    acc = jnp.dot(slo, wsLo_ref[...], preferred_element_type=jnp.float32)
    acc += jnp.dot(shi, wsHi_ref[...], preferred_element_type=jnp.float32)
    acc += jnp.dot(nlo, wnLo_ref[...], preferred_element_type=jnp.float32)
    acc += jnp.dot(nhi, wnHi_ref[...], preferred_element_type=jnp.float32)
    o_ref[...] = jnp.maximum(acc, 0.0)


_BM = 1024


def _tc_combine(self_w, neigh_w, wsLo, wsHi, wnLo, wnHi):
    wspec = pl.BlockSpec((_DW, _E), lambda i: (0, 0))
    return pl.pallas_call(
        _mm_body,
        grid=(_BP // _BM,),
        in_specs=[
            pl.BlockSpec((_BM, _DW), lambda i: (i, 0)),
            pl.BlockSpec((_BM, _DW), lambda i: (i, 0)),
            wspec, wspec, wspec, wspec,
        ],
        out_specs=pl.BlockSpec((_BM, _E), lambda i: (i, 0)),
        out_shape=jax.ShapeDtypeStruct((_BP, _E), jnp.float32),
    )(self_w, neigh_w, wsLo, wsHi, wnLo, wnHi)


def kernel(features, nodes, neigh_idx, W):
    features_w = _pack_table(features)
    nodes_p = jnp.pad(nodes, (0, _BP - _B))
    neigh_p = jnp.pad(neigh_idx, ((0, _BP - _B), (0, 0))).reshape(_BP * _K)
    self_w, neigh_w = _sc_gather(features_w, nodes_p, neigh_p)
    wsT = W[:, :_D].T
    wnT = W[:, _D:].T * (1.0 / _K)
    out = _tc_combine(self_w, neigh_w,
                      wsT[:_DW], wsT[_DW:], wnT[:_DW], wnT[_DW:])
    return out[:_B]
